# baseline (speedup 1.0000x reference)
import functools
import jax
import jax.numpy as jnp
from jax import lax
from jax.experimental import pallas as pl
from jax.experimental.pallas import tpu as pltpu
from jax.experimental.pallas import tpu_sc as plsc

N, DIM = 2048, 768
HEADS, DIM_HEAD = 12, 64
INNER = HEADS * DIM_HEAD
FF = 4 * DIM
NE, TOPK = 8, 2
SHARED_FF = 2 * DIM

EPS = 1e-6

BM = 256
NPAD = TOPK * N + NE * BM
G = NPAD // BM
NW = 32
PW = (TOPK * N) // NW
CH = PW // 2
TW = N // NW


def _dotT(a, b):
    return jax.lax.dot_general(a, b, (((1,), (1,)), ((), ())),
                               preferred_element_type=jnp.float32)


def _silu(v):
    return v * jax.nn.sigmoid(v)


def _ln(v):
    m = jnp.mean(v, axis=-1, keepdims=True)
    c = v - m
    var = jnp.mean(c * c, axis=-1, keepdims=True)
    return c * jax.lax.rsqrt(var + EPS)


def _emb_kernel(t_ref, ada_w_ref, ada_b_ref, emb_ref):
    tt = t_ref[...]
    s = tt * jax.nn.sigmoid(tt)
    emb_ref[...] = _dotT(s, ada_w_ref[...]) + ada_b_ref[...]


def _qkv_kernel(x_ref, emb_ref, wq_ref, wk_ref, wv_ref, bq_ref, bk_ref,
                bv_ref, q_ref, k_ref, v_ref):
    x = x_ref[...]
    shift = emb_ref[:, 0:DIM]
    scale = emb_ref[:, DIM:2 * DIM]
    h = _ln(x) * (1.0 + scale) + shift
    q_ref[...] = (_dotT(h, wq_ref[...]) + bq_ref[...]).astype(jnp.bfloat16)
    k_ref[...] = (_dotT(h, wk_ref[...]) + bk_ref[...]).astype(jnp.bfloat16)
    v_ref[...] = (_dotT(h, wv_ref[...]) + bv_ref[...]).astype(jnp.bfloat16)


def _attn_kernel(q_ref, k_ref, v_ref, o_ref):
    scale = 1.0 / (DIM_HEAD ** 0.5)
    for h in range(HEADS):
        lo, hi = h * DIM_HEAD, (h + 1) * DIM_HEAD
        qh = q_ref[:, lo:hi]
        kh = k_ref[:, lo:hi]
        vh = v_ref[:, lo:hi]
        s = _dotT(qh, kh) * scale
        m = jnp.max(s, axis=-1, keepdims=True)
        p = jnp.exp(s - m)
        p = (p / jnp.sum(p, axis=-1, keepdims=True)).astype(jnp.bfloat16)
        o_ref[:, lo:hi] = jnp.dot(p, vh, preferred_element_type=jnp.float32)


def _post_kernel(o_ref, x_ref, emb_ref, wo_ref, bo_ref, gw_ref,
                 x1_ref, h2_ref, i1_ref, i2_ref, w1_ref, w2_ref):
    attn_out = _dotT(o_ref[...], wo_ref[...]) + bo_ref[...]
    gate_msa = emb_ref[:, 2 * DIM:3 * DIM]
    shift_mlp = emb_ref[:, 3 * DIM:4 * DIM]
    scale_mlp = emb_ref[:, 4 * DIM:5 * DIM]
    x1 = x_ref[...] + gate_msa * attn_out
    x1_ref[...] = x1
    h2 = _ln(x1) * (1.0 + scale_mlp) + shift_mlp
    h2_ref[...] = h2
    logits = _dotT(h2, gw_ref[...])
    mx = jnp.max(logits, axis=-1, keepdims=True)
    ex = jnp.exp(logits - mx)
    s = ex / jnp.sum(ex, axis=-1, keepdims=True)
    bn = s.shape[0]
    e_iota = jax.lax.broadcasted_iota(jnp.int32, (bn, NE), 1)
    w1 = jnp.max(s, axis=-1, keepdims=True)
    i1 = jnp.min(jnp.where(s == w1, e_iota, NE), axis=-1, keepdims=True)
    oh1 = (e_iota == i1)
    s2 = jnp.where(oh1, -1.0, s)
    w2 = jnp.max(s2, axis=-1, keepdims=True)
    i2 = jnp.min(jnp.where(s2 == w2, e_iota, NE), axis=-1, keepdims=True)
    i1_ref[...] = i1
    i2_ref[...] = i2
    w1_ref[...] = w1
    w2_ref[...] = w2


def _route_kernel(i1_ref, i2_ref, pos1_ref, pos2_ref, estep_ref, active_ref):
    i1 = i1_ref[...]
    i2 = i2_ref[...]
    e_iota = jax.lax.broadcasted_iota(jnp.int32, (N, NE), 1)
    oh1 = (i1 == e_iota).astype(jnp.float32)
    oh2 = (i2 == e_iota).astype(jnp.float32)
    ohs = oh1 + oh2
    prev = jnp.zeros((N, NE), jnp.float32)
    CB = 512
    r_iota = jax.lax.broadcasted_iota(jnp.int32, (N, CB), 0)
    c_iota = jax.lax.broadcasted_iota(jnp.int32, (N, CB), 1)
    for c in range(N // CB):
        tri = (c_iota + c * CB < r_iota).astype(jnp.float32)
        prev = prev + jnp.dot(tri, ohs[c * CB:(c + 1) * CB, :],
                              preferred_element_type=jnp.float32)
    counts = jnp.sum(ohs, axis=0, keepdims=True)
    pc = jnp.ceil(counts * (1.0 / BM)) * BM
    e8r = jax.lax.broadcasted_iota(jnp.int32, (NE, NE), 0)
    e8c = jax.lax.broadcasted_iota(jnp.int32, (NE, NE), 1)
    triu = (e8r < e8c).astype(jnp.float32)
    po = jnp.dot(pc, triu, preferred_element_type=jnp.float32)
    r1 = prev
    r2 = prev + oh1
    pos1 = jnp.sum(oh1 * (po + r1), axis=1, keepdims=True)
    pos2 = jnp.sum(oh2 * (po + r2), axis=1, keepdims=True)
    pos1_ref[...] = pos1.astype(jnp.int32)
    pos2_ref[...] = pos2.astype(jnp.int32)
    ends = po + pc
    total = jnp.sum(pc)
    g_vals = jax.lax.broadcasted_iota(
        jnp.int32, (G, 1), 0).astype(jnp.float32) * BM
    estep = jnp.sum((g_vals >= ends).astype(jnp.int32), axis=1, keepdims=True)
    estep_ref[...] = jnp.minimum(estep, NE - 1)
    active_ref[...] = (g_vals < total).astype(jnp.int32)


def _dispatch_kernel(h2_hbm, tok_hbm, pos_hbm, xs_hbm, tok_v, pos_v, buf, sem):
    wid = lax.axis_index("s") * 2 + lax.axis_index("c")
    pltpu.sync_copy(tok_hbm.at[wid], tok_v)
    pltpu.sync_copy(pos_hbm.at[wid], pos_v)
    for j in range(PW // CH):
        pltpu.async_copy(h2_hbm.at[tok_v.at[j]], buf, sem).wait()
        pltpu.async_copy(buf, xs_hbm.at[pos_v.at[j]], sem).wait()


def _gmm_kernel(estep_ref, active_ref, xs_ref, eg_ref, eu_ref, ed_ref, eo_ref):
    g = pl.program_id(0)

    @pl.when(active_ref[g] == 1)
    def _():
        xs = xs_ref[...].astype(jnp.bfloat16)
        eg = eg_ref[0].astype(jnp.bfloat16)
        eu = eu_ref[0].astype(jnp.bfloat16)
        ed = ed_ref[0].astype(jnp.bfloat16)
        a = (_silu(_dotT(xs, eg)) * _dotT(xs, eu)).astype(jnp.bfloat16)
        eo_ref[...] = jax.lax.dot_general(a, ed,
                                          (((1,), (1,)), ((), ())),
                                          preferred_element_type=jnp.float32)


def _combine_kernel(eo_hbm, pa_hbm, pb_hbm, r0_hbm, r1_hbm,
                    ia_v, ib_v, buf, sem):
    wid = lax.axis_index("s") * 2 + lax.axis_index("c")
    pltpu.sync_copy(pa_hbm.at[wid], ia_v)
    pltpu.sync_copy(pb_hbm.at[wid], ib_v)
    pltpu.async_copy(eo_hbm.at[ia_v], buf, sem).wait()
    pltpu.sync_copy(buf, r0_hbm.at[pl.ds(wid * TW, TW)])
    pltpu.async_copy(eo_hbm.at[ib_v], buf, sem).wait()
    pltpu.sync_copy(buf, r1_hbm.at[pl.ds(wid * TW, TW)])


def _final_kernel(h2_ref, r0_ref, r1_ref, w1_ref, w2_ref, x1_ref, emb_ref,
                  sg_ref, su_ref, sd_ref, out_ref):
    h2 = h2_ref[...]
    act = _silu(_dotT(h2, sg_ref[...])) * _dotT(h2, su_ref[...])
    shared = jax.lax.dot_general(act, sd_ref[...], (((1,), (1,)), ((), ())),
                                 preferred_element_type=jnp.float32)
    y = w1_ref[...] * r0_ref[...] + w2_ref[...] * r1_ref[...] + shared
    gate_mlp = emb_ref[:, 5 * DIM:6 * DIM]
    out_ref[...] = x1_ref[...] + gate_mlp * y


def kernel(x, t, ada_w, ada_b, wq, bq, wk, bk, wv, bv, wo, bo, gate_w, eg, eu, ed, sg, su, sd):
    b = x.shape[0]
    xf = x.reshape(N, DIM)
    bq2 = bq.reshape(1, INNER)
    bk2 = bk.reshape(1, INNER)
    bv2 = bv.reshape(1, INNER)
    bo2 = bo.reshape(1, DIM)
    ada_b2 = ada_b.reshape(1, 6 * DIM)

    emb = pl.pallas_call(
        _emb_kernel,
        out_shape=jax.ShapeDtypeStruct((1, 6 * DIM), jnp.float32),
    )(t, ada_w, ada_b2)

    BN = 512
    nb = N // BN
    q, k, v = pl.pallas_call(
        _qkv_kernel,
        grid=(nb,),
        in_specs=[
            pl.BlockSpec((BN, DIM), lambda i: (i, 0)),
            pl.BlockSpec((1, 6 * DIM), lambda i: (0, 0)),
            pl.BlockSpec((INNER, DIM), lambda i: (0, 0)),
            pl.BlockSpec((INNER, DIM), lambda i: (0, 0)),
            pl.BlockSpec((INNER, DIM), lambda i: (0, 0)),
            pl.BlockSpec((1, INNER), lambda i: (0, 0)),
            pl.BlockSpec((1, INNER), lambda i: (0, 0)),
            pl.BlockSpec((1, INNER), lambda i: (0, 0)),
        ],
        out_specs=[
            pl.BlockSpec((BN, INNER), lambda i: (i, 0)),
            pl.BlockSpec((BN, INNER), lambda i: (i, 0)),
            pl.BlockSpec((BN, INNER), lambda i: (i, 0)),
        ],
        out_shape=[jax.ShapeDtypeStruct((N, INNER), jnp.bfloat16)] * 3,
    )(xf, emb, wq, wk, wv, bq2, bk2, bv2)

    BQ = 512
    o = pl.pallas_call(
        _attn_kernel,
        grid=(N // BQ,),
        in_specs=[
            pl.BlockSpec((BQ, INNER), lambda i: (i, 0)),
            pl.BlockSpec((N, INNER), lambda i: (0, 0)),
            pl.BlockSpec((N, INNER), lambda i: (0, 0)),
        ],
        out_specs=pl.BlockSpec((BQ, INNER), lambda i: (i, 0)),
        out_shape=jax.ShapeDtypeStruct((N, INNER), jnp.float32),
    )(q, k, v)

    x1, h2, i1, i2, w1, w2 = pl.pallas_call(
        _post_kernel,
        grid=(nb,),
        in_specs=[
            pl.BlockSpec((BN, INNER), lambda i: (i, 0)),
            pl.BlockSpec((BN, DIM), lambda i: (i, 0)),
            pl.BlockSpec((1, 6 * DIM), lambda i: (0, 0)),
            pl.BlockSpec((DIM, INNER), lambda i: (0, 0)),
            pl.BlockSpec((1, DIM), lambda i: (0, 0)),
            pl.BlockSpec((NE, DIM), lambda i: (0, 0)),
        ],
        out_specs=[
            pl.BlockSpec((BN, DIM), lambda i: (i, 0)),
            pl.BlockSpec((BN, DIM), lambda i: (i, 0)),
            pl.BlockSpec((BN, 1), lambda i: (i, 0)),
            pl.BlockSpec((BN, 1), lambda i: (i, 0)),
            pl.BlockSpec((BN, 1), lambda i: (i, 0)),
            pl.BlockSpec((BN, 1), lambda i: (i, 0)),
        ],
        out_shape=[
            jax.ShapeDtypeStruct((N, DIM), jnp.float32),
            jax.ShapeDtypeStruct((N, DIM), jnp.float32),
            jax.ShapeDtypeStruct((N, 1), jnp.int32),
            jax.ShapeDtypeStruct((N, 1), jnp.int32),
            jax.ShapeDtypeStruct((N, 1), jnp.float32),
            jax.ShapeDtypeStruct((N, 1), jnp.float32),
        ],
    )(o, xf, emb, wo, bo2, gate_w)

    pos1, pos2, estep, active = pl.pallas_call(
        _route_kernel,
        out_shape=[
            jax.ShapeDtypeStruct((N, 1), jnp.int32),
            jax.ShapeDtypeStruct((N, 1), jnp.int32),
            jax.ShapeDtypeStruct((G, 1), jnp.int32),
            jax.ShapeDtypeStruct((G, 1), jnp.int32),
        ],
        compiler_params=pltpu.CompilerParams(
            vmem_limit_bytes=100 * 1024 * 1024),
    )(i1, i2)

    pos_pairs = jnp.concatenate([pos1, pos2], axis=1)
    pos3 = pos_pairs.reshape(NW, PW // CH, CH)
    tok3 = jnp.repeat(jnp.arange(N, dtype=jnp.int32), TOPK).reshape(
        NW, PW // CH, CH)

    mesh = plsc.VectorSubcoreMesh(core_axis_name="c", subcore_axis_name="s")
    xs = pl.kernel(
        _dispatch_kernel,
        mesh=mesh,
        out_type=jax.ShapeDtypeStruct((NPAD, DIM), jnp.float32),
        scratch_types=[
            pltpu.VMEM((PW // CH, CH), jnp.int32),
            pltpu.VMEM((PW // CH, CH), jnp.int32),
            pltpu.VMEM((CH, DIM), jnp.float32),
            pltpu.SemaphoreType.DMA,
        ],
    )(h2, tok3, pos3)

    eo = pl.pallas_call(
        _gmm_kernel,
        grid_spec=pltpu.PrefetchScalarGridSpec(
            num_scalar_prefetch=2,
            grid=(G,),
            in_specs=[
                pl.BlockSpec((BM, DIM), lambda g, es, ac: (g, 0)),
                pl.BlockSpec((1, FF, DIM), lambda g, es, ac: (es[g], 0, 0)),
                pl.BlockSpec((1, FF, DIM), lambda g, es, ac: (es[g], 0, 0)),
                pl.BlockSpec((1, DIM, FF), lambda g, es, ac: (es[g], 0, 0)),
            ],
            out_specs=pl.BlockSpec((BM, DIM), lambda g, es, ac: (g, 0)),
        ),
        out_shape=jax.ShapeDtypeStruct((NPAD, DIM), jnp.float32),
        compiler_params=pltpu.CompilerParams(
            vmem_limit_bytes=110 * 1024 * 1024),
    )(estep.reshape(G), active.reshape(G), xs, eg, eu, ed)

    pa = pos1.reshape(NW, TW)
    pb = pos2.reshape(NW, TW)
    r0, r1 = pl.kernel(
        _combine_kernel,
        mesh=mesh,
        out_type=[
            jax.ShapeDtypeStruct((N, DIM), jnp.float32),
            jax.ShapeDtypeStruct((N, DIM), jnp.float32),
        ],
        scratch_types=[
            pltpu.VMEM((TW,), jnp.int32),
            pltpu.VMEM((TW,), jnp.int32),
            pltpu.VMEM((TW, DIM), jnp.float32),
            pltpu.SemaphoreType.DMA,
        ],
    )(eo, pa, pb)

    out = pl.pallas_call(
        _final_kernel,
        grid=(nb,),
        in_specs=[
            pl.BlockSpec((BN, DIM), lambda i: (i, 0)),
            pl.BlockSpec((BN, DIM), lambda i: (i, 0)),
            pl.BlockSpec((BN, DIM), lambda i: (i, 0)),
            pl.BlockSpec((BN, 1), lambda i: (i, 0)),
            pl.BlockSpec((BN, 1), lambda i: (i, 0)),
            pl.BlockSpec((BN, DIM), lambda i: (i, 0)),
            pl.BlockSpec((1, 6 * DIM), lambda i: (0, 0)),
            pl.BlockSpec((SHARED_FF, DIM), lambda i: (0, 0)),
            pl.BlockSpec((SHARED_FF, DIM), lambda i: (0, 0)),
            pl.BlockSpec((DIM, SHARED_FF), lambda i: (0, 0)),
        ],
        out_specs=pl.BlockSpec((BN, DIM), lambda i: (i, 0)),
        out_shape=jax.ShapeDtypeStruct((N, DIM), jnp.float32),
    )(h2, r0, r1, w1, w2, x1, emb, sg, su, sd)

    return out.reshape(b, N, DIM)


# speedup vs baseline: 1.0913x; 1.0913x over previous
import functools
import jax
import jax.numpy as jnp
from jax import lax
from jax.experimental import pallas as pl
from jax.experimental.pallas import tpu as pltpu
from jax.experimental.pallas import tpu_sc as plsc

N, DIM = 2048, 768
HEADS, DIM_HEAD = 12, 64
INNER = HEADS * DIM_HEAD
FF = 4 * DIM
NE, TOPK = 8, 2
SHARED_FF = 2 * DIM

EPS = 1e-6

BM = 256
NPAD = TOPK * N + NE * BM
G = NPAD // BM
NW = 32
PW = (TOPK * N) // NW
CH = PW // 2
TW = N // NW


def _dotT(a, b):
    return jax.lax.dot_general(a, b, (((1,), (1,)), ((), ())),
                               preferred_element_type=jnp.float32)


def _silu(v):
    return v * jax.nn.sigmoid(v)


def _ln(v):
    m = jnp.mean(v, axis=-1, keepdims=True)
    c = v - m
    var = jnp.mean(c * c, axis=-1, keepdims=True)
    return c * jax.lax.rsqrt(var + EPS)


def _emb_kernel(t_ref, ada_w_ref, ada_b_ref, emb_ref):
    tt = t_ref[...]
    s = tt * jax.nn.sigmoid(tt)
    emb_ref[...] = _dotT(s, ada_w_ref[...]) + ada_b_ref[...]


def _qkv_kernel(x_ref, emb_ref, wq_ref, wk_ref, wv_ref, bq_ref, bk_ref,
                bv_ref, q_ref, k_ref, v_ref):
    x = x_ref[...]
    shift = emb_ref[:, 0:DIM]
    scale = emb_ref[:, DIM:2 * DIM]
    h = _ln(x) * (1.0 + scale) + shift
    q_ref[...] = (_dotT(h, wq_ref[...]) + bq_ref[...]).astype(jnp.bfloat16)
    k_ref[...] = (_dotT(h, wk_ref[...]) + bk_ref[...]).astype(jnp.bfloat16)
    v_ref[...] = (_dotT(h, wv_ref[...]) + bv_ref[...]).astype(jnp.bfloat16)


def _attn_kernel(q_ref, k_ref, v_ref, o_ref):
    scale = 1.0 / (DIM_HEAD ** 0.5)
    for h in range(HEADS):
        lo, hi = h * DIM_HEAD, (h + 1) * DIM_HEAD
        qh = q_ref[:, lo:hi]
        kh = k_ref[:, lo:hi]
        vh = v_ref[:, lo:hi]
        s = _dotT(qh, kh) * scale
        m = jnp.max(s, axis=-1, keepdims=True)
        p = jnp.exp(s - m)
        denom = jnp.sum(p, axis=-1, keepdims=True)
        pv = jnp.dot(p.astype(jnp.bfloat16), vh,
                     preferred_element_type=jnp.float32)
        o_ref[:, lo:hi] = pv / denom


def _post_kernel(o_ref, x_ref, emb_ref, wo_ref, bo_ref, gw_ref,
                 x1_ref, h2_ref, i1_ref, i2_ref, w1_ref, w2_ref):
    attn_out = _dotT(o_ref[...], wo_ref[...]) + bo_ref[...]
    gate_msa = emb_ref[:, 2 * DIM:3 * DIM]
    shift_mlp = emb_ref[:, 3 * DIM:4 * DIM]
    scale_mlp = emb_ref[:, 4 * DIM:5 * DIM]
    x1 = x_ref[...] + gate_msa * attn_out
    x1_ref[...] = x1
    h2 = _ln(x1) * (1.0 + scale_mlp) + shift_mlp
    h2_ref[...] = h2
    logits = _dotT(h2, gw_ref[...])
    mx = jnp.max(logits, axis=-1, keepdims=True)
    ex = jnp.exp(logits - mx)
    s = ex / jnp.sum(ex, axis=-1, keepdims=True)
    bn = s.shape[0]
    e_iota = jax.lax.broadcasted_iota(jnp.int32, (bn, NE), 1)
    w1 = jnp.max(s, axis=-1, keepdims=True)
    i1 = jnp.min(jnp.where(s == w1, e_iota, NE), axis=-1, keepdims=True)
    oh1 = (e_iota == i1)
    s2 = jnp.where(oh1, -1.0, s)
    w2 = jnp.max(s2, axis=-1, keepdims=True)
    i2 = jnp.min(jnp.where(s2 == w2, e_iota, NE), axis=-1, keepdims=True)
    i1_ref[...] = i1
    i2_ref[...] = i2
    w1_ref[...] = w1
    w2_ref[...] = w2


def _route_kernel(i1_ref, i2_ref, pos1_ref, pos2_ref, estep_ref, active_ref):
    i1 = i1_ref[...]
    i2 = i2_ref[...]
    e_iota = jax.lax.broadcasted_iota(jnp.int32, (N, NE), 1)
    oh1 = (i1 == e_iota).astype(jnp.float32)
    oh2 = (i2 == e_iota).astype(jnp.float32)
    ohs = oh1 + oh2
    prev = jnp.zeros((N, NE), jnp.float32)
    CB = 512
    r_iota = jax.lax.broadcasted_iota(jnp.int32, (N, CB), 0)
    c_iota = jax.lax.broadcasted_iota(jnp.int32, (N, CB), 1)
    for c in range(N // CB):
        tri = (c_iota + c * CB < r_iota).astype(jnp.float32)
        prev = prev + jnp.dot(tri, ohs[c * CB:(c + 1) * CB, :],
                              preferred_element_type=jnp.float32)
    counts = jnp.sum(ohs, axis=0, keepdims=True)
    pc = jnp.ceil(counts * (1.0 / BM)) * BM
    e8r = jax.lax.broadcasted_iota(jnp.int32, (NE, NE), 0)
    e8c = jax.lax.broadcasted_iota(jnp.int32, (NE, NE), 1)
    triu = (e8r < e8c).astype(jnp.float32)
    po = jnp.dot(pc, triu, preferred_element_type=jnp.float32)
    r1 = prev
    r2 = prev + oh1
    pos1 = jnp.sum(oh1 * (po + r1), axis=1, keepdims=True)
    pos2 = jnp.sum(oh2 * (po + r2), axis=1, keepdims=True)
    pos1_ref[...] = pos1.astype(jnp.int32)
    pos2_ref[...] = pos2.astype(jnp.int32)
    ends = po + pc
    total = jnp.sum(pc)
    g_vals = jax.lax.broadcasted_iota(
        jnp.int32, (G, 1), 0).astype(jnp.float32) * BM
    estep = jnp.sum((g_vals >= ends).astype(jnp.int32), axis=1, keepdims=True)
    estep_ref[...] = jnp.minimum(estep, NE - 1)
    active_ref[...] = (g_vals < total).astype(jnp.int32)


def _dispatch_kernel(h2_hbm, tok_hbm, pos_hbm, xs_hbm, tok_v, pos_v, buf, sem):
    wid = lax.axis_index("s") * 2 + lax.axis_index("c")
    pltpu.sync_copy(tok_hbm.at[wid], tok_v)
    pltpu.sync_copy(pos_hbm.at[wid], pos_v)
    for j in range(PW // CH):
        pltpu.async_copy(h2_hbm.at[tok_v.at[j]], buf, sem).wait()
        pltpu.async_copy(buf, xs_hbm.at[pos_v.at[j]], sem).wait()


def _gmm_kernel(estep_ref, active_ref, xs_ref, eg_ref, eu_ref, ed_ref, eo_ref):
    g = pl.program_id(0)

    @pl.when(active_ref[g] == 1)
    def _():
        xs = xs_ref[...].astype(jnp.bfloat16)
        eg = eg_ref[0].astype(jnp.bfloat16)
        eu = eu_ref[0].astype(jnp.bfloat16)
        ed = ed_ref[0].astype(jnp.bfloat16)
        a = (_silu(_dotT(xs, eg)) * _dotT(xs, eu)).astype(jnp.bfloat16)
        eo_ref[...] = jax.lax.dot_general(a, ed,
                                          (((1,), (1,)), ((), ())),
                                          preferred_element_type=jnp.float32)


def _combine_kernel(eo_hbm, pa_hbm, pb_hbm, r0_hbm, r1_hbm,
                    ia_v, ib_v, buf, sem):
    wid = lax.axis_index("s") * 2 + lax.axis_index("c")
    pltpu.sync_copy(pa_hbm.at[wid], ia_v)
    pltpu.sync_copy(pb_hbm.at[wid], ib_v)
    pltpu.async_copy(eo_hbm.at[ia_v], buf, sem).wait()
    pltpu.sync_copy(buf, r0_hbm.at[pl.ds(wid * TW, TW)])
    pltpu.async_copy(eo_hbm.at[ib_v], buf, sem).wait()
    pltpu.sync_copy(buf, r1_hbm.at[pl.ds(wid * TW, TW)])


def _final_kernel(h2_ref, r0_ref, r1_ref, w1_ref, w2_ref, x1_ref, emb_ref,
                  sg_ref, su_ref, sd_ref, out_ref):
    h2 = h2_ref[...]
    act = _silu(_dotT(h2, sg_ref[...])) * _dotT(h2, su_ref[...])
    shared = jax.lax.dot_general(act, sd_ref[...], (((1,), (1,)), ((), ())),
                                 preferred_element_type=jnp.float32)
    y = w1_ref[...] * r0_ref[...] + w2_ref[...] * r1_ref[...] + shared
    gate_mlp = emb_ref[:, 5 * DIM:6 * DIM]
    out_ref[...] = x1_ref[...] + gate_mlp * y


def kernel(x, t, ada_w, ada_b, wq, bq, wk, bk, wv, bv, wo, bo, gate_w, eg, eu, ed, sg, su, sd):
    b = x.shape[0]
    xf = x.reshape(N, DIM)
    bq2 = bq.reshape(1, INNER)
    bk2 = bk.reshape(1, INNER)
    bv2 = bv.reshape(1, INNER)
    bo2 = bo.reshape(1, DIM)
    ada_b2 = ada_b.reshape(1, 6 * DIM)

    emb = pl.pallas_call(
        _emb_kernel,
        out_shape=jax.ShapeDtypeStruct((1, 6 * DIM), jnp.float32),
    )(t, ada_w, ada_b2)

    BN = 512
    nb = N // BN
    q, k, v = pl.pallas_call(
        _qkv_kernel,
        grid=(nb,),
        in_specs=[
            pl.BlockSpec((BN, DIM), lambda i: (i, 0)),
            pl.BlockSpec((1, 6 * DIM), lambda i: (0, 0)),
            pl.BlockSpec((INNER, DIM), lambda i: (0, 0)),
            pl.BlockSpec((INNER, DIM), lambda i: (0, 0)),
            pl.BlockSpec((INNER, DIM), lambda i: (0, 0)),
            pl.BlockSpec((1, INNER), lambda i: (0, 0)),
            pl.BlockSpec((1, INNER), lambda i: (0, 0)),
            pl.BlockSpec((1, INNER), lambda i: (0, 0)),
        ],
        out_specs=[
            pl.BlockSpec((BN, INNER), lambda i: (i, 0)),
            pl.BlockSpec((BN, INNER), lambda i: (i, 0)),
            pl.BlockSpec((BN, INNER), lambda i: (i, 0)),
        ],
        out_shape=[jax.ShapeDtypeStruct((N, INNER), jnp.bfloat16)] * 3,
    )(xf, emb, wq, wk, wv, bq2, bk2, bv2)

    BQ = 512
    o = pl.pallas_call(
        _attn_kernel,
        grid=(N // BQ,),
        in_specs=[
            pl.BlockSpec((BQ, INNER), lambda i: (i, 0)),
            pl.BlockSpec((N, INNER), lambda i: (0, 0)),
            pl.BlockSpec((N, INNER), lambda i: (0, 0)),
        ],
        out_specs=pl.BlockSpec((BQ, INNER), lambda i: (i, 0)),
        out_shape=jax.ShapeDtypeStruct((N, INNER), jnp.float32),
    )(q, k, v)

    x1, h2, i1, i2, w1, w2 = pl.pallas_call(
        _post_kernel,
        grid=(nb,),
        in_specs=[
            pl.BlockSpec((BN, INNER), lambda i: (i, 0)),
            pl.BlockSpec((BN, DIM), lambda i: (i, 0)),
            pl.BlockSpec((1, 6 * DIM), lambda i: (0, 0)),
            pl.BlockSpec((DIM, INNER), lambda i: (0, 0)),
            pl.BlockSpec((1, DIM), lambda i: (0, 0)),
            pl.BlockSpec((NE, DIM), lambda i: (0, 0)),
        ],
        out_specs=[
            pl.BlockSpec((BN, DIM), lambda i: (i, 0)),
            pl.BlockSpec((BN, DIM), lambda i: (i, 0)),
            pl.BlockSpec((BN, 1), lambda i: (i, 0)),
            pl.BlockSpec((BN, 1), lambda i: (i, 0)),
            pl.BlockSpec((BN, 1), lambda i: (i, 0)),
            pl.BlockSpec((BN, 1), lambda i: (i, 0)),
        ],
        out_shape=[
            jax.ShapeDtypeStruct((N, DIM), jnp.float32),
            jax.ShapeDtypeStruct((N, DIM), jnp.float32),
            jax.ShapeDtypeStruct((N, 1), jnp.int32),
            jax.ShapeDtypeStruct((N, 1), jnp.int32),
            jax.ShapeDtypeStruct((N, 1), jnp.float32),
            jax.ShapeDtypeStruct((N, 1), jnp.float32),
        ],
    )(o, xf, emb, wo, bo2, gate_w)

    pos1, pos2, estep, active = pl.pallas_call(
        _route_kernel,
        out_shape=[
            jax.ShapeDtypeStruct((N, 1), jnp.int32),
            jax.ShapeDtypeStruct((N, 1), jnp.int32),
            jax.ShapeDtypeStruct((G, 1), jnp.int32),
            jax.ShapeDtypeStruct((G, 1), jnp.int32),
        ],
        compiler_params=pltpu.CompilerParams(
            vmem_limit_bytes=100 * 1024 * 1024),
    )(i1, i2)

    pos_pairs = jnp.concatenate([pos1, pos2], axis=1)
    pos3 = pos_pairs.reshape(NW, PW // CH, CH)
    tok3 = jnp.repeat(jnp.arange(N, dtype=jnp.int32), TOPK).reshape(
        NW, PW // CH, CH)

    mesh = plsc.VectorSubcoreMesh(core_axis_name="c", subcore_axis_name="s")
    xs = pl.kernel(
        _dispatch_kernel,
        mesh=mesh,
        out_type=jax.ShapeDtypeStruct((NPAD, DIM), jnp.float32),
        scratch_types=[
            pltpu.VMEM((PW // CH, CH), jnp.int32),
            pltpu.VMEM((PW // CH, CH), jnp.int32),
            pltpu.VMEM((CH, DIM), jnp.float32),
            pltpu.SemaphoreType.DMA,
        ],
    )(h2, tok3, pos3)

    eo = pl.pallas_call(
        _gmm_kernel,
        grid_spec=pltpu.PrefetchScalarGridSpec(
            num_scalar_prefetch=2,
            grid=(G,),
            in_specs=[
                pl.BlockSpec((BM, DIM), lambda g, es, ac: (g, 0)),
                pl.BlockSpec((1, FF, DIM), lambda g, es, ac: (es[g], 0, 0)),
                pl.BlockSpec((1, FF, DIM), lambda g, es, ac: (es[g], 0, 0)),
                pl.BlockSpec((1, DIM, FF), lambda g, es, ac: (es[g], 0, 0)),
            ],
            out_specs=pl.BlockSpec((BM, DIM), lambda g, es, ac: (g, 0)),
        ),
        out_shape=jax.ShapeDtypeStruct((NPAD, DIM), jnp.float32),
        compiler_params=pltpu.CompilerParams(
            vmem_limit_bytes=110 * 1024 * 1024),
    )(estep.reshape(G), active.reshape(G), xs, eg, eu, ed)

    pa = pos1.reshape(NW, TW)
    pb = pos2.reshape(NW, TW)
    r0, r1 = pl.kernel(
        _combine_kernel,
        mesh=mesh,
        out_type=[
            jax.ShapeDtypeStruct((N, DIM), jnp.float32),
            jax.ShapeDtypeStruct((N, DIM), jnp.float32),
        ],
        scratch_types=[
            pltpu.VMEM((TW,), jnp.int32),
            pltpu.VMEM((TW,), jnp.int32),
            pltpu.VMEM((TW, DIM), jnp.float32),
            pltpu.SemaphoreType.DMA,
        ],
    )(eo, pa, pb)

    out = pl.pallas_call(
        _final_kernel,
        grid=(nb,),
        in_specs=[
            pl.BlockSpec((BN, DIM), lambda i: (i, 0)),
            pl.BlockSpec((BN, DIM), lambda i: (i, 0)),
            pl.BlockSpec((BN, DIM), lambda i: (i, 0)),
            pl.BlockSpec((BN, 1), lambda i: (i, 0)),
            pl.BlockSpec((BN, 1), lambda i: (i, 0)),
            pl.BlockSpec((BN, DIM), lambda i: (i, 0)),
            pl.BlockSpec((1, 6 * DIM), lambda i: (0, 0)),
            pl.BlockSpec((SHARED_FF, DIM), lambda i: (0, 0)),
            pl.BlockSpec((SHARED_FF, DIM), lambda i: (0, 0)),
            pl.BlockSpec((DIM, SHARED_FF), lambda i: (0, 0)),
        ],
        out_specs=pl.BlockSpec((BN, DIM), lambda i: (i, 0)),
        out_shape=jax.ShapeDtypeStruct((N, DIM), jnp.float32),
    )(h2, r0, r1, w1, w2, x1, emb, sg, su, sd)

    return out.reshape(b, N, DIM)


# speedup vs baseline: 1.1238x; 1.0297x over previous
import functools
import jax
import jax.numpy as jnp
from jax import lax
from jax.experimental import pallas as pl
from jax.experimental.pallas import tpu as pltpu
from jax.experimental.pallas import tpu_sc as plsc

N, DIM = 2048, 768
HEADS, DIM_HEAD = 12, 64
INNER = HEADS * DIM_HEAD
FF = 4 * DIM
NE, TOPK = 8, 2
SHARED_FF = 2 * DIM

EPS = 1e-6

BM = 256
NPAD = TOPK * N + NE * BM
G = NPAD // BM
NW = 32
PW = (TOPK * N) // NW
CH = PW // 2
TW = N // NW


def _dotT(a, b):
    return jax.lax.dot_general(a, b, (((1,), (1,)), ((), ())),
                               preferred_element_type=jnp.float32)


def _silu(v):
    return v * jax.nn.sigmoid(v)


def _ln(v):
    m = jnp.mean(v, axis=-1, keepdims=True)
    c = v - m
    var = jnp.mean(c * c, axis=-1, keepdims=True)
    return c * jax.lax.rsqrt(var + EPS)


def _emb_kernel(t_ref, ada_w_ref, ada_b_ref, emb_ref):
    tt = t_ref[...]
    s = tt * jax.nn.sigmoid(tt)
    emb_ref[...] = _dotT(s, ada_w_ref[...]) + ada_b_ref[...]


def _qkv_kernel(x_ref, emb_ref, wq_ref, wk_ref, wv_ref, bq_ref, bk_ref,
                bv_ref, q_ref, k_ref, v_ref):
    x = x_ref[...]
    shift = emb_ref[:, 0:DIM]
    scale = emb_ref[:, DIM:2 * DIM]
    h = _ln(x) * (1.0 + scale) + shift
    q_ref[...] = ((_dotT(h, wq_ref[...]) + bq_ref[...])
                  * (1.0 / (DIM_HEAD ** 0.5))).astype(jnp.bfloat16)
    k_ref[...] = (_dotT(h, wk_ref[...]) + bk_ref[...]).astype(jnp.bfloat16)
    v_ref[...] = (_dotT(h, wv_ref[...]) + bv_ref[...]).astype(jnp.bfloat16)


def _attn_kernel(q_ref, k_ref, v_ref, o_ref):
    for h in range(HEADS):
        lo, hi = h * DIM_HEAD, (h + 1) * DIM_HEAD
        qh = q_ref[:, lo:hi]
        kh = k_ref[:, lo:hi]
        vh = v_ref[:, lo:hi]
        s = _dotT(qh, kh)
        m = jnp.max(s, axis=-1, keepdims=True)
        p = jnp.exp(s - m)
        denom = jnp.sum(p, axis=-1, keepdims=True)
        pv = jnp.dot(p.astype(jnp.bfloat16), vh,
                     preferred_element_type=jnp.float32)
        o_ref[:, lo:hi] = pv / denom


def _post_kernel(o_ref, x_ref, emb_ref, wo_ref, bo_ref, gw_ref,
                 x1_ref, h2_ref, i1_ref, i2_ref, w1_ref, w2_ref):
    attn_out = _dotT(o_ref[...], wo_ref[...]) + bo_ref[...]
    gate_msa = emb_ref[:, 2 * DIM:3 * DIM]
    shift_mlp = emb_ref[:, 3 * DIM:4 * DIM]
    scale_mlp = emb_ref[:, 4 * DIM:5 * DIM]
    x1 = x_ref[...] + gate_msa * attn_out
    x1_ref[...] = x1
    h2 = _ln(x1) * (1.0 + scale_mlp) + shift_mlp
    h2_ref[...] = h2
    logits = _dotT(h2, gw_ref[...])
    mx = jnp.max(logits, axis=-1, keepdims=True)
    ex = jnp.exp(logits - mx)
    s = ex / jnp.sum(ex, axis=-1, keepdims=True)
    bn = s.shape[0]
    e_iota = jax.lax.broadcasted_iota(jnp.int32, (bn, NE), 1)
    w1 = jnp.max(s, axis=-1, keepdims=True)
    i1 = jnp.min(jnp.where(s == w1, e_iota, NE), axis=-1, keepdims=True)
    oh1 = (e_iota == i1)
    s2 = jnp.where(oh1, -1.0, s)
    w2 = jnp.max(s2, axis=-1, keepdims=True)
    i2 = jnp.min(jnp.where(s2 == w2, e_iota, NE), axis=-1, keepdims=True)
    i1_ref[...] = i1
    i2_ref[...] = i2
    w1_ref[...] = w1
    w2_ref[...] = w2


def _route_kernel(i1_ref, i2_ref, pos1_ref, pos2_ref, estep_ref, active_ref):
    i1 = i1_ref[...]
    i2 = i2_ref[...]
    e_iota = jax.lax.broadcasted_iota(jnp.int32, (N, NE), 1)
    oh1 = (i1 == e_iota).astype(jnp.float32)
    oh2 = (i2 == e_iota).astype(jnp.float32)
    ohs = oh1 + oh2
    prev = jnp.zeros((N, NE), jnp.float32)
    CB = 512
    r_iota = jax.lax.broadcasted_iota(jnp.int32, (N, CB), 0)
    c_iota = jax.lax.broadcasted_iota(jnp.int32, (N, CB), 1)
    for c in range(N // CB):
        tri = (c_iota + c * CB < r_iota).astype(jnp.float32)
        prev = prev + jnp.dot(tri, ohs[c * CB:(c + 1) * CB, :],
                              preferred_element_type=jnp.float32)
    counts = jnp.sum(ohs, axis=0, keepdims=True)
    pc = jnp.ceil(counts * (1.0 / BM)) * BM
    e8r = jax.lax.broadcasted_iota(jnp.int32, (NE, NE), 0)
    e8c = jax.lax.broadcasted_iota(jnp.int32, (NE, NE), 1)
    triu = (e8r < e8c).astype(jnp.float32)
    po = jnp.dot(pc, triu, preferred_element_type=jnp.float32)
    r1 = prev
    r2 = prev + oh1
    pos1 = jnp.sum(oh1 * (po + r1), axis=1, keepdims=True)
    pos2 = jnp.sum(oh2 * (po + r2), axis=1, keepdims=True)
    pos1_ref[...] = pos1.astype(jnp.int32)
    pos2_ref[...] = pos2.astype(jnp.int32)
    ends = po + pc
    total = jnp.sum(pc)
    g_vals = jax.lax.broadcasted_iota(
        jnp.int32, (G, 1), 0).astype(jnp.float32) * BM
    estep = jnp.sum((g_vals >= ends).astype(jnp.int32), axis=1, keepdims=True)
    estep_ref[...] = jnp.minimum(estep, NE - 1)
    active_ref[...] = (g_vals < total).astype(jnp.int32)


def _dispatch_kernel(h2_hbm, tok_hbm, pos_hbm, xs_hbm, tok_v, pos_v, buf, sem):
    wid = lax.axis_index("s") * 2 + lax.axis_index("c")
    pltpu.sync_copy(tok_hbm.at[wid], tok_v)
    pltpu.sync_copy(pos_hbm.at[wid], pos_v)
    for j in range(PW // CH):
        pltpu.async_copy(h2_hbm.at[tok_v.at[j]], buf, sem).wait()
        pltpu.async_copy(buf, xs_hbm.at[pos_v.at[j]], sem).wait()


def _gmm_kernel(estep_ref, active_ref, xs_ref, eg_ref, eu_ref, ed_ref, eo_ref):
    g = pl.program_id(0)

    @pl.when(active_ref[g] == 1)
    def _():
        xs = xs_ref[...].astype(jnp.bfloat16)
        eg = eg_ref[0].astype(jnp.bfloat16)
        eu = eu_ref[0].astype(jnp.bfloat16)
        ed = ed_ref[0].astype(jnp.bfloat16)
        a = (_silu(_dotT(xs, eg)) * _dotT(xs, eu)).astype(jnp.bfloat16)
        eo_ref[...] = jax.lax.dot_general(a, ed,
                                          (((1,), (1,)), ((), ())),
                                          preferred_element_type=jnp.float32)


def _combine_kernel(eo_hbm, pa_hbm, pb_hbm, r0_hbm, r1_hbm,
                    ia_v, ib_v, buf, sem):
    wid = lax.axis_index("s") * 2 + lax.axis_index("c")
    pltpu.sync_copy(pa_hbm.at[wid], ia_v)
    pltpu.sync_copy(pb_hbm.at[wid], ib_v)
    pltpu.async_copy(eo_hbm.at[ia_v], buf, sem).wait()
    pltpu.sync_copy(buf, r0_hbm.at[pl.ds(wid * TW, TW)])
    pltpu.async_copy(eo_hbm.at[ib_v], buf, sem).wait()
    pltpu.sync_copy(buf, r1_hbm.at[pl.ds(wid * TW, TW)])


def _final_kernel(h2_ref, r0_ref, r1_ref, w1_ref, w2_ref, x1_ref, emb_ref,
                  sg_ref, su_ref, sd_ref, out_ref):
    h2 = h2_ref[...]
    act = _silu(_dotT(h2, sg_ref[...])) * _dotT(h2, su_ref[...])
    shared = jax.lax.dot_general(act, sd_ref[...], (((1,), (1,)), ((), ())),
                                 preferred_element_type=jnp.float32)
    y = w1_ref[...] * r0_ref[...] + w2_ref[...] * r1_ref[...] + shared
    gate_mlp = emb_ref[:, 5 * DIM:6 * DIM]
    out_ref[...] = x1_ref[...] + gate_mlp * y


def kernel(x, t, ada_w, ada_b, wq, bq, wk, bk, wv, bv, wo, bo, gate_w, eg, eu, ed, sg, su, sd):
    b = x.shape[0]
    xf = x.reshape(N, DIM)
    bq2 = bq.reshape(1, INNER)
    bk2 = bk.reshape(1, INNER)
    bv2 = bv.reshape(1, INNER)
    bo2 = bo.reshape(1, DIM)
    ada_b2 = ada_b.reshape(1, 6 * DIM)

    emb = pl.pallas_call(
        _emb_kernel,
        out_shape=jax.ShapeDtypeStruct((1, 6 * DIM), jnp.float32),
    )(t, ada_w, ada_b2)

    BN = 512
    nb = N // BN
    q, k, v = pl.pallas_call(
        _qkv_kernel,
        grid=(nb,),
        in_specs=[
            pl.BlockSpec((BN, DIM), lambda i: (i, 0)),
            pl.BlockSpec((1, 6 * DIM), lambda i: (0, 0)),
            pl.BlockSpec((INNER, DIM), lambda i: (0, 0)),
            pl.BlockSpec((INNER, DIM), lambda i: (0, 0)),
            pl.BlockSpec((INNER, DIM), lambda i: (0, 0)),
            pl.BlockSpec((1, INNER), lambda i: (0, 0)),
            pl.BlockSpec((1, INNER), lambda i: (0, 0)),
            pl.BlockSpec((1, INNER), lambda i: (0, 0)),
        ],
        out_specs=[
            pl.BlockSpec((BN, INNER), lambda i: (i, 0)),
            pl.BlockSpec((BN, INNER), lambda i: (i, 0)),
            pl.BlockSpec((BN, INNER), lambda i: (i, 0)),
        ],
        out_shape=[jax.ShapeDtypeStruct((N, INNER), jnp.bfloat16)] * 3,
    )(xf, emb, wq, wk, wv, bq2, bk2, bv2)

    BQ = 512
    o = pl.pallas_call(
        _attn_kernel,
        grid=(N // BQ,),
        in_specs=[
            pl.BlockSpec((BQ, INNER), lambda i: (i, 0)),
            pl.BlockSpec((N, INNER), lambda i: (0, 0)),
            pl.BlockSpec((N, INNER), lambda i: (0, 0)),
        ],
        out_specs=pl.BlockSpec((BQ, INNER), lambda i: (i, 0)),
        out_shape=jax.ShapeDtypeStruct((N, INNER), jnp.float32),
    )(q, k, v)

    x1, h2, i1, i2, w1, w2 = pl.pallas_call(
        _post_kernel,
        grid=(nb,),
        in_specs=[
            pl.BlockSpec((BN, INNER), lambda i: (i, 0)),
            pl.BlockSpec((BN, DIM), lambda i: (i, 0)),
            pl.BlockSpec((1, 6 * DIM), lambda i: (0, 0)),
            pl.BlockSpec((DIM, INNER), lambda i: (0, 0)),
            pl.BlockSpec((1, DIM), lambda i: (0, 0)),
            pl.BlockSpec((NE, DIM), lambda i: (0, 0)),
        ],
        out_specs=[
            pl.BlockSpec((BN, DIM), lambda i: (i, 0)),
            pl.BlockSpec((BN, DIM), lambda i: (i, 0)),
            pl.BlockSpec((BN, 1), lambda i: (i, 0)),
            pl.BlockSpec((BN, 1), lambda i: (i, 0)),
            pl.BlockSpec((BN, 1), lambda i: (i, 0)),
            pl.BlockSpec((BN, 1), lambda i: (i, 0)),
        ],
        out_shape=[
            jax.ShapeDtypeStruct((N, DIM), jnp.float32),
            jax.ShapeDtypeStruct((N, DIM), jnp.float32),
            jax.ShapeDtypeStruct((N, 1), jnp.int32),
            jax.ShapeDtypeStruct((N, 1), jnp.int32),
            jax.ShapeDtypeStruct((N, 1), jnp.float32),
            jax.ShapeDtypeStruct((N, 1), jnp.float32),
        ],
    )(o, xf, emb, wo, bo2, gate_w)

    pos1, pos2, estep, active = pl.pallas_call(
        _route_kernel,
        out_shape=[
            jax.ShapeDtypeStruct((N, 1), jnp.int32),
            jax.ShapeDtypeStruct((N, 1), jnp.int32),
            jax.ShapeDtypeStruct((G, 1), jnp.int32),
            jax.ShapeDtypeStruct((G, 1), jnp.int32),
        ],
        compiler_params=pltpu.CompilerParams(
            vmem_limit_bytes=100 * 1024 * 1024),
    )(i1, i2)

    pos_pairs = jnp.concatenate([pos1, pos2], axis=1)
    pos3 = pos_pairs.reshape(NW, PW // CH, CH)
    tok3 = jnp.repeat(jnp.arange(N, dtype=jnp.int32), TOPK).reshape(
        NW, PW // CH, CH)

    mesh = plsc.VectorSubcoreMesh(core_axis_name="c", subcore_axis_name="s")
    xs = pl.kernel(
        _dispatch_kernel,
        mesh=mesh,
        out_type=jax.ShapeDtypeStruct((NPAD, DIM), jnp.float32),
        scratch_types=[
            pltpu.VMEM((PW // CH, CH), jnp.int32),
            pltpu.VMEM((PW // CH, CH), jnp.int32),
            pltpu.VMEM((CH, DIM), jnp.float32),
            pltpu.SemaphoreType.DMA,
        ],
    )(h2, tok3, pos3)

    eo = pl.pallas_call(
        _gmm_kernel,
        grid_spec=pltpu.PrefetchScalarGridSpec(
            num_scalar_prefetch=2,
            grid=(G,),
            in_specs=[
                pl.BlockSpec((BM, DIM), lambda g, es, ac: (g, 0)),
                pl.BlockSpec((1, FF, DIM), lambda g, es, ac: (es[g], 0, 0)),
                pl.BlockSpec((1, FF, DIM), lambda g, es, ac: (es[g], 0, 0)),
                pl.BlockSpec((1, DIM, FF), lambda g, es, ac: (es[g], 0, 0)),
            ],
            out_specs=pl.BlockSpec((BM, DIM), lambda g, es, ac: (g, 0)),
        ),
        out_shape=jax.ShapeDtypeStruct((NPAD, DIM), jnp.float32),
        compiler_params=pltpu.CompilerParams(
            vmem_limit_bytes=110 * 1024 * 1024),
    )(estep.reshape(G), active.reshape(G), xs, eg, eu, ed)

    pa = pos1.reshape(NW, TW)
    pb = pos2.reshape(NW, TW)
    r0, r1 = pl.kernel(
        _combine_kernel,
        mesh=mesh,
        out_type=[
            jax.ShapeDtypeStruct((N, DIM), jnp.float32),
            jax.ShapeDtypeStruct((N, DIM), jnp.float32),
        ],
        scratch_types=[
            pltpu.VMEM((TW,), jnp.int32),
            pltpu.VMEM((TW,), jnp.int32),
            pltpu.VMEM((TW, DIM), jnp.float32),
            pltpu.SemaphoreType.DMA,
        ],
    )(eo, pa, pb)

    out = pl.pallas_call(
        _final_kernel,
        grid=(nb,),
        in_specs=[
            pl.BlockSpec((BN, DIM), lambda i: (i, 0)),
            pl.BlockSpec((BN, DIM), lambda i: (i, 0)),
            pl.BlockSpec((BN, DIM), lambda i: (i, 0)),
            pl.BlockSpec((BN, 1), lambda i: (i, 0)),
            pl.BlockSpec((BN, 1), lambda i: (i, 0)),
            pl.BlockSpec((BN, DIM), lambda i: (i, 0)),
            pl.BlockSpec((1, 6 * DIM), lambda i: (0, 0)),
            pl.BlockSpec((SHARED_FF, DIM), lambda i: (0, 0)),
            pl.BlockSpec((SHARED_FF, DIM), lambda i: (0, 0)),
            pl.BlockSpec((DIM, SHARED_FF), lambda i: (0, 0)),
        ],
        out_specs=pl.BlockSpec((BN, DIM), lambda i: (i, 0)),
        out_shape=jax.ShapeDtypeStruct((N, DIM), jnp.float32),
    )(h2, r0, r1, w1, w2, x1, emb, sg, su, sd)

    return out.reshape(b, N, DIM)
